# Initial kernel scaffold; baseline (speedup 1.0000x reference)
import jax
import jax.numpy as jnp
from jax.experimental import pallas as pl


def kernel(x, table, W, b):
    raise NotImplementedError("write your pallas kernel here")


# speedup vs baseline: 1.6241x; 1.6241x over previous
import functools

import jax
import jax.numpy as jnp
from jax import lax
from jax.experimental import pallas as pl
from jax.experimental.pallas import tpu as pltpu
from jax.experimental.pallas import tpu_sc as plsc

BATCH = 4096
SEQ = 200
SEQ_PAD = 208
EMBED_DIM = 32
NUM_CLASSES = 100
CLASS_PAD = 128

NUM_CORES = 2
NUM_SUBCORES = 16
NUM_WORKERS = NUM_CORES * NUM_SUBCORES
BPW = BATCH // NUM_WORKERS

_N16 = SEQ_PAD // 16
_G1 = 128
_G2 = SEQ_PAD - _G1


def _sc_pool_sums(x_pad, table):
    mesh = plsc.VectorSubcoreMesh(core_axis_name="c", subcore_axis_name="s")

    @functools.partial(
        pl.kernel,
        mesh=mesh,
        compiler_params=pltpu.CompilerParams(use_tc_tiling_on_sc=False),
        out_type=jax.ShapeDtypeStruct((BATCH, EMBED_DIM), jnp.float32),
        scratch_types=[
            pltpu.VMEM((BPW, SEQ_PAD), jnp.int32),
            pltpu.VMEM((SEQ_PAD, EMBED_DIM), jnp.float32),
            pltpu.VMEM((BPW, EMBED_DIM), jnp.float32),
            pltpu.SemaphoreType.DMA,
        ],
    )
    def body(x_hbm, table_hbm, out_hbm, idx_v, rows_v, sums_v, sem):
        wid = lax.axis_index("s") * NUM_CORES + lax.axis_index("c")
        base = wid * BPW
        pltpu.sync_copy(x_hbm.at[pl.ds(base, BPW)], idx_v)

        def row_body(b, carry):
            cp1 = pltpu.async_copy(
                table_hbm.at[idx_v.at[b, pl.ds(0, _G1)]],
                rows_v.at[pl.ds(0, _G1)], sem)
            cp2 = pltpu.async_copy(
                table_hbm.at[idx_v.at[b, pl.ds(_G1, _G2)]],
                rows_v.at[pl.ds(_G1, _G2)], sem)
            cp1.wait()
            cp2.wait()

            zero = jnp.zeros((16,), jnp.float32)

            def chunk(c, accs):
                a0, a1, a2, a3, a4, a5, a6, a7 = accs
                r0 = c * 16
                for j in range(0, 16, 4):
                    a0 = a0 + rows_v[r0 + j, pl.ds(0, 16)]
                    a1 = a1 + rows_v[r0 + j, pl.ds(16, 16)]
                    a2 = a2 + rows_v[r0 + j + 1, pl.ds(0, 16)]
                    a3 = a3 + rows_v[r0 + j + 1, pl.ds(16, 16)]
                    a4 = a4 + rows_v[r0 + j + 2, pl.ds(0, 16)]
                    a5 = a5 + rows_v[r0 + j + 2, pl.ds(16, 16)]
                    a6 = a6 + rows_v[r0 + j + 3, pl.ds(0, 16)]
                    a7 = a7 + rows_v[r0 + j + 3, pl.ds(16, 16)]
                return (a0, a1, a2, a3, a4, a5, a6, a7)

            accs = lax.fori_loop(0, _N16, chunk, (zero,) * 8)

            s0 = (accs[0] + accs[2]) + (accs[4] + accs[6])
            s1 = (accs[1] + accs[3]) + (accs[5] + accs[7])
            sums_v[b, pl.ds(0, 16)] = s0
            sums_v[b, pl.ds(16, 16)] = s1
            return carry

        lax.fori_loop(0, BPW, row_body, 0)
        pltpu.sync_copy(sums_v, out_hbm.at[pl.ds(base, BPW)])

    return body(x_pad, table)


def _tc_matmul(sums, x_pad, t0, w_scaled, b_pad):
    def body(s_ref, x_ref, t0_ref, w_ref, b_ref, o_ref):
        cnt0 = jnp.sum((x_ref[...] == 0).astype(jnp.float32), axis=1,
                       keepdims=True)
        pooled = s_ref[...] - cnt0 * t0_ref[...]
        o_ref[...] = jnp.dot(
            pooled, w_ref[...], preferred_element_type=jnp.float32
        ) + b_ref[...]

    blk = 1024
    return pl.pallas_call(
        body,
        grid=(BATCH // blk,),
        in_specs=[
            pl.BlockSpec((blk, EMBED_DIM), lambda i: (i, 0)),
            pl.BlockSpec((blk, SEQ_PAD), lambda i: (i, 0)),
            pl.BlockSpec((1, EMBED_DIM), lambda i: (0, 0)),
            pl.BlockSpec((EMBED_DIM, CLASS_PAD), lambda i: (0, 0)),
            pl.BlockSpec((1, CLASS_PAD), lambda i: (0, 0)),
        ],
        out_specs=pl.BlockSpec((blk, CLASS_PAD), lambda i: (i, 0)),
        out_shape=jax.ShapeDtypeStruct((BATCH, CLASS_PAD), jnp.float32),
    )(sums, x_pad, t0, w_scaled, b_pad)


def kernel(x, table, W, b):
    x_pad = jnp.pad(x, ((0, 0), (0, SEQ_PAD - SEQ)))
    t0 = lax.slice(table, (0, 0), (1, EMBED_DIM))
    w_scaled = jnp.zeros((EMBED_DIM, CLASS_PAD), jnp.float32)
    w_scaled = w_scaled.at[:, :NUM_CLASSES].set(W.T * (1.0 / SEQ))
    b_pad = jnp.zeros((1, CLASS_PAD), jnp.float32).at[0, :NUM_CLASSES].set(b)

    sums = _sc_pool_sums(x_pad, table)
    logits_pad = _tc_matmul(sums, x_pad, t0, w_scaled, b_pad)
    return logits_pad[:, :NUM_CLASSES]
